# baseline (speedup 1.0000x reference)
import functools
import math

import jax
import jax.numpy as jnp
from jax import lax
from jax.experimental import pallas as pl
from jax.experimental.pallas import tpu as pltpu
from jax.experimental.pallas import tpu_sc as plsc

_NUM_CORES = 2
_NUM_SUBCORES = 16
_NUM_WORKERS = _NUM_CORES * _NUM_SUBCORES
_LANES = 16
_BT = 128
_SW = 133
_NBUF = 3


@functools.lru_cache(maxsize=None)
def _make_lookup(V, D, B0, S, W):
    assert B0 == _BT * _NUM_WORKERS, "one output tile-column per worker"
    assert D % 8 == 0 and W >= D
    DT = D // 8
    scale = float(math.sqrt(D))
    mesh = plsc.VectorSubcoreMesh(core_axis_name="c", subcore_axis_name="s")

    @functools.partial(
        pl.kernel,
        mesh=mesh,
        out_type=jax.ShapeDtypeStruct((S, DT, _NUM_WORKERS, 8, _BT),
                                      jnp.float32),
        scratch_types=[
            pltpu.VMEM((S, _BT), jnp.int32),
            pltpu.VMEM((_NBUF * _BT, W), jnp.float32),
            pltpu.VMEM((1, _NBUF * D, _SW), jnp.float32),
            pltpu.SemaphoreType.DMA,
            pltpu.SemaphoreType.DMA,
        ],
        compiler_params=pltpu.CompilerParams(
            use_tc_tiling_on_sc=False, needs_layout_passes=False
        ),
    )
    def lookup(table_hbm, idxt_hbm, out_hbm, idx_v, rows, stage, gsem, ssem):
        wid = lax.axis_index("s") * _NUM_CORES + lax.axis_index("c")
        pltpu.sync_copy(idxt_hbm.at[:, pl.ds(wid * _BT, _BT)], idx_v)

        iota = lax.broadcasted_iota(jnp.int32, (_LANES,), 0)
        zerov = jnp.zeros((_LANES,), jnp.int32)

        def gather_copy(s, slot):
            return pltpu.make_async_copy(
                table_hbm.at[idx_v.at[s]],
                rows.at[pl.ds(slot * _BT, _BT)],
                gsem,
            )

        def scatter_copies(s, slot):
            return [
                pltpu.make_async_copy(
                    stage.at[0, pl.ds(slot * D + dt * 8, 8), pl.ds(0, _BT)],
                    out_hbm.at[s, dt, wid],
                    ssem,
                )
                for dt in range(DT)
            ]

        def transpose_scale(slot):
            dvs = [iota + (slot * D + c * _LANES) for c in range(D // _LANES)]

            @plsc.parallel_loop(0, _BT, 1, unroll=4)
            def _(r):
                rv = zerov + r
                row = slot * _BT + r
                for c in range(D // _LANES):
                    v = rows[row, pl.ds(c * _LANES, _LANES)]
                    plsc.store_scatter(
                        stage, [zerov, dvs[c], rv], v * scale
                    )

        gather_copy(0, 0).start()
        gather_copy(1, 1).start()

        def body(s, carry):
            slot = lax.rem(s, _NBUF)

            @pl.when(s >= 2)
            def _():
                for c in scatter_copies(s - 2, lax.rem(s - 2, _NBUF)):
                    c.wait()

            @pl.when(s + 2 <= S - 1)
            def _():
                gather_copy(s + 2, lax.rem(s + 2, _NBUF)).start()

            gather_copy(s, slot).wait()
            transpose_scale(slot)
            for c in scatter_copies(s, slot):
                c.start()
            return carry

        lax.fori_loop(0, S, body, 0)
        for c in scatter_copies(S - 2, lax.rem(S - 2, _NBUF)):
            c.wait()
        for c in scatter_copies(S - 1, lax.rem(S - 1, _NBUF)):
            c.wait()

    return lookup


def kernel(word_indices, embedding_matrix):
    B0, S = word_indices.shape
    V, D = embedding_matrix.shape
    idxt = word_indices.T.astype(jnp.int32)
    W = D if D % 8 == 0 and D % 128 != 64 else D + 8
    table = embedding_matrix
    if W != D:
        table = jnp.pad(embedding_matrix, ((0, 0), (0, W - D)))
    lookup = _make_lookup(V, D, B0, S, W)
    out5 = lookup(table, idxt)
    out = out5.transpose(2, 4, 0, 1, 3).reshape(B0, S, D)
    return out


# speedup vs baseline: 1.8218x; 1.8218x over previous
import functools
import math

import jax
import jax.numpy as jnp
from jax import lax
from jax.experimental import pallas as pl
from jax.experimental.pallas import tpu as pltpu
from jax.experimental.pallas import tpu_sc as plsc

_NUM_CORES = 2
_NUM_SUBCORES = 16
_NUM_WORKERS = _NUM_CORES * _NUM_SUBCORES
_LANES = 16
_BT = 128
_SW = 133
_NBUF = 3


@functools.lru_cache(maxsize=None)
def _make_lookup(V, D, B0, S, W):
    assert B0 == _BT * _NUM_WORKERS, "one output tile-column per worker"
    assert D % 8 == 0 and W >= D
    DT = D // 8
    scale = float(math.sqrt(D))
    mesh = plsc.VectorSubcoreMesh(core_axis_name="c", subcore_axis_name="s")

    @functools.partial(
        pl.kernel,
        mesh=mesh,
        out_type=jax.ShapeDtypeStruct((S, DT, _NUM_WORKERS, 8, _BT),
                                      jnp.float32),
        scratch_types=[
            pltpu.VMEM((S, _BT), jnp.int32),
            pltpu.VMEM((_NBUF * _BT, W), jnp.float32),
            pltpu.VMEM((1, _NBUF * D, _SW), jnp.float32),
            pltpu.SemaphoreType.DMA,
            pltpu.SemaphoreType.DMA,
        ],
        compiler_params=pltpu.CompilerParams(
            use_tc_tiling_on_sc=False, needs_layout_passes=False
        ),
    )
    def lookup(table_hbm, idxt_hbm, out_hbm, idx_v, rows, stage, gsem, ssem):
        wid = lax.axis_index("s") * _NUM_CORES + lax.axis_index("c")
        pltpu.sync_copy(idxt_hbm.at[:, pl.ds(wid * _BT, _BT)], idx_v)

        iota = lax.broadcasted_iota(jnp.int32, (_LANES,), 0)
        zerov = jnp.zeros((_LANES,), jnp.int32)

        def gather_copy(s, slot):
            return pltpu.make_async_copy(
                table_hbm.at[idx_v.at[s]],
                rows.at[pl.ds(slot * _BT, _BT)],
                gsem,
            )

        def scatter_copies(s, slot):
            return [
                pltpu.make_async_copy(
                    stage.at[0, pl.ds(slot * D + dt * 8, 8), pl.ds(0, _BT)],
                    out_hbm.at[s, dt, wid],
                    ssem,
                )
                for dt in range(DT)
            ]

        def transpose_scale(slot):
            dvs = [iota + (slot * D + c * _LANES) for c in range(D // _LANES)]

            @plsc.parallel_loop(0, _BT, 1, unroll=4)
            def _(r):
                rv = zerov + r
                row = slot * _BT + r
                for c in range(D // _LANES):
                    v = rows[row, pl.ds(c * _LANES, _LANES)]
                    plsc.store_scatter(
                        stage, [zerov, dvs[c], rv], v * scale
                    )

        gather_copy(0, 0).start()
        gather_copy(1, 1).start()

        def body(s, carry):
            slot = lax.rem(s, _NBUF)

            @pl.when(s >= 2)
            def _():
                for c in scatter_copies(s - 2, lax.rem(s - 2, _NBUF)):
                    c.wait()

            @pl.when(s + 2 <= S - 1)
            def _():
                gather_copy(s + 2, lax.rem(s + 2, _NBUF)).start()

            gather_copy(s, slot).wait()
            transpose_scale(slot)
            for c in scatter_copies(s, slot):
                c.start()
            return carry

        lax.fori_loop(0, S, body, 0)
        for c in scatter_copies(S - 2, lax.rem(S - 2, _NBUF)):
            c.wait()
        for c in scatter_copies(S - 1, lax.rem(S - 1, _NBUF)):
            c.wait()

    return lookup


def kernel(word_indices, embedding_matrix):
    B0, S = word_indices.shape
    V, D = embedding_matrix.shape
    idxt = word_indices.T.astype(jnp.int32)
    assert D == 64, "kernel specialized for 128-word padded row pitch"
    table = jnp.pad(embedding_matrix, ((0, 0), (0, 128 - D)))
    table2 = table.reshape(2 * V, D)
    idxt = idxt * 2
    lookup = _make_lookup(2 * V, D, B0, S, D)
    out5 = lookup(table2, idxt)
    out = out5.transpose(2, 4, 0, 1, 3).reshape(B0, S, D)
    return out
